# baseline (speedup 1.0000x reference)
import functools

import jax
import jax.numpy as jnp
from jax import lax
from jax.experimental import pallas as pl
from jax.experimental.pallas import tpu as pltpu
from jax.experimental.pallas import tpu_sc as plsc

B, T_IN, F_IN = 5000, 10, 4
HID, OUT_F, T_OUT = 64, 2, 106
N = B * T_IN
E = 800000

NC, NS = 2, 16
NW = NC * NS
N_PAD = 50176
STRIPE = N_PAD // NS
E_PAD = 851968
EROWS = E_PAD // 128
ROWS_PER_TILE = EROWS // NW
G_IN = 16
G_OUT = ROWS_PER_TILE // G_IN
WR = 8

_mesh = plsc.VectorSubcoreMesh(
    core_axis_name="c", subcore_axis_name="s", num_cores=NC, num_subcores=NS)
_sc_params = pltpu.CompilerParams(use_tc_tiling_on_sc=False)

f32 = jnp.float32
i32 = jnp.int32


def _deg_kernel(dst2d, ones_hbm, zeros_hbm, out0, out1, dstbuf, ones_v, acc,
                sem):
    c = lax.axis_index("c")
    s = lax.axis_index("s")
    w = c * NS + s
    r0 = s * STRIPE
    pltpu.sync_copy(zeros_hbm.at[pl.ds(r0, STRIPE)], acc.at[pl.ds(r0, STRIPE)])
    pltpu.sync_copy(ones_hbm, ones_v)
    plsc.subcore_barrier()
    base = w * ROWS_PER_TILE

    def body(g, carry):
        row = base + g * G_IN
        pltpu.sync_copy(dst2d.at[pl.ds(row, G_IN)], dstbuf)
        for j in range(G_IN):
            pltpu.sync_copy(ones_v, acc.at[dstbuf.at[j]], add=True)
        return carry

    lax.fori_loop(0, G_OUT, body, 0)
    plsc.subcore_barrier()

    @pl.when(c == 0)
    def _():
        pltpu.sync_copy(acc.at[pl.ds(r0, STRIPE)], out0.at[pl.ds(r0, STRIPE)])

    @pl.when(c == 1)
    def _():
        pltpu.sync_copy(acc.at[pl.ds(r0, STRIPE)], out1.at[pl.ds(r0, STRIPE)])


_deg_call = functools.partial(
    pl.kernel,
    out_type=[jax.ShapeDtypeStruct((N_PAD, WR), f32)] * NC,
    mesh=_mesh,
    compiler_params=_sc_params,
    scratch_types=[
        pltpu.VMEM((G_IN, 128), i32),
        pltpu.VMEM((128, WR), f32),
        pltpu.VMEM_SHARED((N_PAD, WR), f32),
        pltpu.SemaphoreType.DMA,
    ],
)(_deg_kernel)


def _make_scatter_call(K):
    def body(src2d, dst2d, zeros_hbm, *rest):
        tables = rest[:K]
        outs = rest[K:K + NC * K]
        srcbuf, dstbuf, rows, tab_spm, acc, sem, sem2 = rest[K + NC * K:]
        c = lax.axis_index("c")
        s = lax.axis_index("s")
        w = c * NS + s
        r0 = s * STRIPE
        base = w * ROWS_PER_TILE
        for kc in range(K):
            pltpu.sync_copy(zeros_hbm.at[pl.ds(r0, STRIPE)],
                            acc.at[pl.ds(r0, STRIPE)])
            pltpu.sync_copy(tables[kc].at[pl.ds(r0, STRIPE)],
                            tab_spm.at[pl.ds(r0, STRIPE)])
            plsc.subcore_barrier()

            def inner(g, carry):
                row = base + g * G_IN
                pltpu.sync_copy(src2d.at[pl.ds(row, G_IN)], srcbuf)
                pltpu.sync_copy(dst2d.at[pl.ds(row, G_IN)], dstbuf)
                cps = [pltpu.async_copy(tab_spm.at[srcbuf.at[j]], rows.at[j],
                                        sem) for j in range(G_IN)]
                for cp in cps:
                    cp.wait()
                cps2 = [pltpu.async_copy(rows.at[j], acc.at[dstbuf.at[j]],
                                         sem2, add=True) for j in range(G_IN)]
                for cp in cps2:
                    cp.wait()
                return carry

            lax.fori_loop(0, G_OUT, inner, 0)
            plsc.subcore_barrier()

            @pl.when(c == 0)
            def _():
                pltpu.sync_copy(acc.at[pl.ds(r0, STRIPE)],
                                outs[kc].at[pl.ds(r0, STRIPE)])

            @pl.when(c == 1)
            def _():
                pltpu.sync_copy(acc.at[pl.ds(r0, STRIPE)],
                                outs[K + kc].at[pl.ds(r0, STRIPE)])

            plsc.subcore_barrier()

    return functools.partial(
        pl.kernel,
        out_type=[jax.ShapeDtypeStruct((N_PAD, WR), f32)] * (NC * K),
        mesh=_mesh,
        compiler_params=_sc_params,
        scratch_types=[
            pltpu.VMEM((G_IN, 128), i32),
            pltpu.VMEM((G_IN, 128), i32),
            pltpu.VMEM((G_IN, 128, WR), f32),
            pltpu.VMEM_SHARED((N_PAD, WR), f32),
            pltpu.VMEM_SHARED((N_PAD, WR), f32),
            pltpu.SemaphoreType.DMA,
            pltpu.SemaphoreType.DMA,
        ],
    )(body)


_scatter1_call = _make_scatter_call(1)


def _t0_body(degp0_ref, degp1_ref, x16_ref, dinv_ref, u1_ref):
    deg = degp0_ref[:, 0] + degp1_ref[:, 0] + 1.0
    dinv = 1.0 / jnp.sqrt(deg)
    dinv_ref[...] = dinv[:, None]
    u1_ref[...] = x16_ref[...] * dinv[:, None]


def _t1_body(acc10_ref, acc11_ref, u1_ref, dinv_ref, w1_ref, b1_ref,
             w2_ref, wo_ref, v2_ref):
    dinv = dinv_ref[:, 0]
    g = (acc10_ref[...] + acc11_ref[...] + u1_ref[...])[:, :F_IN] * dinv[:, None]
    h1 = jnp.dot(g, w1_ref[...], preferred_element_type=f32,
                 precision=lax.Precision.HIGHEST) + b1_ref[...][None, :]
    u2 = jnp.maximum(h1, 0.0) * dinv[:, None]
    w2o = jnp.dot(w2_ref[...], wo_ref[...], preferred_element_type=f32,
                  precision=lax.Precision.HIGHEST)
    v2 = jnp.dot(u2, w2o, preferred_element_type=f32,
                 precision=lax.Precision.HIGHEST)
    v2_ref[...] = jnp.concatenate(
        [v2, jnp.zeros((v2.shape[0], WR - OUT_F), f32)], axis=1)


def _t2_body(a20, a21, v2_ref, dinv_ref, b2_ref, wo_ref, z_ref):
    dinv = dinv_ref[:, 0]
    bz = jnp.dot(b2_ref[...][None, :], wo_ref[...],
                 preferred_element_type=f32,
                 precision=lax.Precision.HIGHEST)
    g = (a20[...] + a21[...] + v2_ref[...])[:, :OUT_F] * dinv[:, None]
    z_ref[...] = g + bz


def _t3_body(zr_ref, wp_ref, bp_ref, out_ref):
    out_ref[...] = jnp.dot(zr_ref[...], wp_ref[...], preferred_element_type=f32,
                           precision=lax.Precision.HIGHEST) + bp_ref[...][None, :]


def kernel(x, edge_index, W1, b1, W2, b2, Wt, bt, Wo, bo):
    pad = jnp.full((2, E_PAD - E), N, dtype=edge_index.dtype)
    ei = jnp.concatenate([edge_index, pad], axis=1)
    src2d = ei[0].reshape(EROWS, 128)
    dst2d = ei[1].reshape(EROWS, 128)
    x16 = jnp.zeros((N_PAD, WR), f32).at[:N, :F_IN].set(x.reshape(N, F_IN))
    zeros16 = jnp.zeros((N_PAD, WR), f32)
    ones16 = jnp.ones((128, WR), f32)
    eye2 = jnp.eye(OUT_F, dtype=f32)
    Wp = (Wt[:, None, :, None] * eye2[None, :, None, :]).reshape(
        T_IN * OUT_F, T_OUT * OUT_F)
    s_wo = Wo.sum(axis=0)
    bp = (bt[:, None] * s_wo[None, :] + bo[None, :]).reshape(-1)

    degp0, degp1 = _deg_call(dst2d, ones16, zeros16)

    nblk = NS
    full2d = lambda shape: pl.BlockSpec(shape, lambda i: (0, 0))
    rowblk = lambda w_: pl.BlockSpec((STRIPE, w_), lambda i: (i, 0))
    dinv, u1 = pl.pallas_call(
        _t0_body,
        grid=(nblk,),
        in_specs=[rowblk(WR), rowblk(WR), rowblk(WR)],
        out_specs=[rowblk(1), rowblk(WR)],
        out_shape=[
            jax.ShapeDtypeStruct((N_PAD, 1), f32),
            jax.ShapeDtypeStruct((N_PAD, WR), f32),
        ],
    )(degp0, degp1, x16)

    acc10, acc11 = _scatter1_call(src2d, dst2d, zeros16, u1)

    v2 = pl.pallas_call(
        _t1_body,
        grid=(nblk,),
        in_specs=[rowblk(WR), rowblk(WR), rowblk(WR), rowblk(1),
                  full2d((F_IN, HID)), pl.BlockSpec((HID,), lambda i: (0,)),
                  full2d((HID, HID)), full2d((HID, OUT_F))],
        out_specs=rowblk(WR),
        out_shape=jax.ShapeDtypeStruct((N_PAD, WR), f32),
    )(acc10, acc11, u1, dinv, W1, b1, W2, Wo)

    acc20, acc21 = _scatter1_call(src2d, dst2d, zeros16, v2)

    zblk = 2000
    zrow = lambda w_: pl.BlockSpec((zblk, w_), lambda i: (i, 0))
    z = pl.pallas_call(
        _t2_body,
        grid=(N // zblk,),
        in_specs=[zrow(WR), zrow(WR), zrow(WR), zrow(1),
                  pl.BlockSpec((HID,), lambda i: (0,)),
                  full2d((HID, OUT_F))],
        out_specs=pl.BlockSpec((zblk, OUT_F), lambda i: (i, 0)),
        out_shape=jax.ShapeDtypeStruct((N, OUT_F), f32),
    )(acc20, acc21, v2, dinv, b2, Wo)

    zr = z.reshape(B, T_IN * OUT_F)
    out2d = pl.pallas_call(
        _t3_body,
        grid=(1,),
        in_specs=[
            full2d((B, T_IN * OUT_F)),
            full2d((T_IN * OUT_F, T_OUT * OUT_F)),
            pl.BlockSpec((T_OUT * OUT_F,), lambda i: (0,)),
        ],
        out_specs=full2d((B, T_OUT * OUT_F)),
        out_shape=jax.ShapeDtypeStruct((B, T_OUT * OUT_F), f32),
    )(zr, Wp, bp)
    return out2d.reshape(B, T_OUT, OUT_F)


# speedup vs baseline: 1.0751x; 1.0751x over previous
import functools

import jax
import jax.numpy as jnp
from jax import lax
from jax.experimental import pallas as pl
from jax.experimental.pallas import tpu as pltpu
from jax.experimental.pallas import tpu_sc as plsc

B, T_IN, F_IN = 5000, 10, 4
HID, OUT_F, T_OUT = 64, 2, 106
N = B * T_IN
E = 800000

NC, NS = 2, 16
NW = NC * NS
N_PAD = 50176
STRIPE = N_PAD // NS
E_PAD = 819200
EROWS = E_PAD // 128
ROWS_PER_TILE = EROWS // NW
G_IN = 8
G_OUT = ROWS_PER_TILE // G_IN
WR = 8

_mesh = plsc.VectorSubcoreMesh(
    core_axis_name="c", subcore_axis_name="s", num_cores=NC, num_subcores=NS)
_sc_params = pltpu.CompilerParams(use_tc_tiling_on_sc=False)

f32 = jnp.float32
i32 = jnp.int32


def _deg_kernel(dst2d, ones_hbm, zeros_hbm, out0, out1, dstbuf, ones_v, acc,
                sem):
    c = lax.axis_index("c")
    s = lax.axis_index("s")
    w = c * NS + s
    r0 = s * STRIPE
    pltpu.sync_copy(zeros_hbm.at[pl.ds(r0, STRIPE)], acc.at[pl.ds(r0, STRIPE)])
    pltpu.sync_copy(ones_hbm, ones_v)
    plsc.subcore_barrier()
    base = w * ROWS_PER_TILE

    def body(g, carry):
        row = base + g * G_IN
        pltpu.sync_copy(dst2d.at[pl.ds(row, G_IN)], dstbuf)
        for j in range(G_IN):
            pltpu.sync_copy(ones_v, acc.at[dstbuf.at[j]], add=True)
        return carry

    lax.fori_loop(0, G_OUT, body, 0)
    plsc.subcore_barrier()

    @pl.when(c == 0)
    def _():
        pltpu.sync_copy(acc.at[pl.ds(r0, STRIPE)], out0.at[pl.ds(r0, STRIPE)])

    @pl.when(c == 1)
    def _():
        pltpu.sync_copy(acc.at[pl.ds(r0, STRIPE)], out1.at[pl.ds(r0, STRIPE)])


_deg_call = functools.partial(
    pl.kernel,
    out_type=[jax.ShapeDtypeStruct((N_PAD, WR), f32)] * NC,
    mesh=_mesh,
    compiler_params=_sc_params,
    scratch_types=[
        pltpu.VMEM((G_IN, 128), i32),
        pltpu.VMEM((128, WR), f32),
        pltpu.VMEM_SHARED((N_PAD, WR), f32),
        pltpu.SemaphoreType.DMA,
    ],
)(_deg_kernel)


def _make_scatter_call(K):
    def body(src2d, dst2d, zeros_hbm, *rest):
        tables = rest[:K]
        outs = rest[K:K + NC * K]
        srcbuf, dstbuf, rows, tab_spm, acc, sem, sem2 = rest[K + NC * K:]
        c = lax.axis_index("c")
        s = lax.axis_index("s")
        w = c * NS + s
        r0 = s * STRIPE
        base = w * ROWS_PER_TILE
        for kc in range(K):
            pltpu.sync_copy(zeros_hbm.at[pl.ds(r0, STRIPE)],
                            acc.at[pl.ds(r0, STRIPE)])
            pltpu.sync_copy(tables[kc].at[pl.ds(r0, STRIPE)],
                            tab_spm.at[pl.ds(r0, STRIPE)])
            plsc.subcore_barrier()

            def inner(g, carry):
                row = base + g * G_IN
                pltpu.sync_copy(src2d.at[pl.ds(row, G_IN)], srcbuf)
                pltpu.sync_copy(dst2d.at[pl.ds(row, G_IN)], dstbuf)
                cps = [pltpu.async_copy(tab_spm.at[srcbuf.at[j]], rows.at[j],
                                        sem) for j in range(G_IN)]
                for cp in cps:
                    cp.wait()
                cps2 = [pltpu.async_copy(rows.at[j], acc.at[dstbuf.at[j]],
                                         sem2, add=True) for j in range(G_IN)]
                for cp in cps2:
                    cp.wait()
                return carry

            lax.fori_loop(0, G_OUT, inner, 0)
            plsc.subcore_barrier()

            @pl.when(c == 0)
            def _():
                pltpu.sync_copy(acc.at[pl.ds(r0, STRIPE)],
                                outs[kc].at[pl.ds(r0, STRIPE)])

            @pl.when(c == 1)
            def _():
                pltpu.sync_copy(acc.at[pl.ds(r0, STRIPE)],
                                outs[K + kc].at[pl.ds(r0, STRIPE)])

            plsc.subcore_barrier()

    return functools.partial(
        pl.kernel,
        out_type=[jax.ShapeDtypeStruct((N_PAD, WR), f32)] * (NC * K),
        mesh=_mesh,
        compiler_params=_sc_params,
        scratch_types=[
            pltpu.VMEM((G_IN, 128), i32),
            pltpu.VMEM((G_IN, 128), i32),
            pltpu.VMEM((G_IN, 128, WR), f32),
            pltpu.VMEM_SHARED((N_PAD, WR), f32),
            pltpu.VMEM_SHARED((N_PAD, WR), f32),
            pltpu.SemaphoreType.DMA,
            pltpu.SemaphoreType.DMA,
        ],
    )(body)


_scatter1_call = _make_scatter_call(1)


def _t0_body(degp0_ref, degp1_ref, x16_ref, dinv_ref, u1_ref):
    deg = degp0_ref[:, 0] + degp1_ref[:, 0] + 1.0
    dinv = 1.0 / jnp.sqrt(deg)
    dinv_ref[...] = dinv[:, None]
    u1_ref[...] = x16_ref[...] * dinv[:, None]


def _t1_body(acc10_ref, acc11_ref, u1_ref, dinv_ref, w1_ref, b1_ref,
             w2_ref, wo_ref, v2_ref):
    dinv = dinv_ref[:, 0]
    g = (acc10_ref[...] + acc11_ref[...] + u1_ref[...])[:, :F_IN] * dinv[:, None]
    h1 = jnp.dot(g, w1_ref[...], preferred_element_type=f32,
                 precision=lax.Precision.HIGHEST) + b1_ref[...][None, :]
    u2 = jnp.maximum(h1, 0.0) * dinv[:, None]
    w2o = jnp.dot(w2_ref[...], wo_ref[...], preferred_element_type=f32,
                  precision=lax.Precision.HIGHEST)
    v2 = jnp.dot(u2, w2o, preferred_element_type=f32,
                 precision=lax.Precision.HIGHEST)
    v2_ref[...] = jnp.concatenate(
        [v2, jnp.zeros((v2.shape[0], WR - OUT_F), f32)], axis=1)


def _t2_body(a20, a21, v2_ref, dinv_ref, b2_ref, wo_ref, z_ref):
    dinv = dinv_ref[:, 0]
    bz = jnp.dot(b2_ref[...][None, :], wo_ref[...],
                 preferred_element_type=f32,
                 precision=lax.Precision.HIGHEST)
    g = (a20[...] + a21[...] + v2_ref[...])[:, :OUT_F] * dinv[:, None]
    z_ref[...] = g + bz


def _t3_body(zr_ref, wp_ref, bp_ref, out_ref):
    out_ref[...] = jnp.dot(zr_ref[...], wp_ref[...], preferred_element_type=f32,
                           precision=lax.Precision.HIGHEST) + bp_ref[...][None, :]


def kernel(x, edge_index, W1, b1, W2, b2, Wt, bt, Wo, bo):
    pad = jnp.full((2, E_PAD - E), N, dtype=edge_index.dtype)
    ei = jnp.concatenate([edge_index, pad], axis=1)
    src2d = ei[0].reshape(EROWS, 128)
    dst2d = ei[1].reshape(EROWS, 128)
    x16 = jnp.zeros((N_PAD, WR), f32).at[:N, :F_IN].set(x.reshape(N, F_IN))
    zeros16 = jnp.zeros((N_PAD, WR), f32)
    ones16 = jnp.ones((128, WR), f32)
    eye2 = jnp.eye(OUT_F, dtype=f32)
    Wp = (Wt[:, None, :, None] * eye2[None, :, None, :]).reshape(
        T_IN * OUT_F, T_OUT * OUT_F)
    s_wo = Wo.sum(axis=0)
    bp = (bt[:, None] * s_wo[None, :] + bo[None, :]).reshape(-1)

    degp0, degp1 = _deg_call(dst2d, ones16, zeros16)

    nblk = NS
    full2d = lambda shape: pl.BlockSpec(shape, lambda i: (0, 0))
    rowblk = lambda w_: pl.BlockSpec((STRIPE, w_), lambda i: (i, 0))
    dinv, u1 = pl.pallas_call(
        _t0_body,
        grid=(nblk,),
        in_specs=[rowblk(WR), rowblk(WR), rowblk(WR)],
        out_specs=[rowblk(1), rowblk(WR)],
        out_shape=[
            jax.ShapeDtypeStruct((N_PAD, 1), f32),
            jax.ShapeDtypeStruct((N_PAD, WR), f32),
        ],
    )(degp0, degp1, x16)

    acc10, acc11 = _scatter1_call(src2d, dst2d, zeros16, u1)

    v2 = pl.pallas_call(
        _t1_body,
        grid=(nblk,),
        in_specs=[rowblk(WR), rowblk(WR), rowblk(WR), rowblk(1),
                  full2d((F_IN, HID)), pl.BlockSpec((HID,), lambda i: (0,)),
                  full2d((HID, HID)), full2d((HID, OUT_F))],
        out_specs=rowblk(WR),
        out_shape=jax.ShapeDtypeStruct((N_PAD, WR), f32),
    )(acc10, acc11, u1, dinv, W1, b1, W2, Wo)

    acc20, acc21 = _scatter1_call(src2d, dst2d, zeros16, v2)

    zblk = 2000
    zrow = lambda w_: pl.BlockSpec((zblk, w_), lambda i: (i, 0))
    z = pl.pallas_call(
        _t2_body,
        grid=(N // zblk,),
        in_specs=[zrow(WR), zrow(WR), zrow(WR), zrow(1),
                  pl.BlockSpec((HID,), lambda i: (0,)),
                  full2d((HID, OUT_F))],
        out_specs=pl.BlockSpec((zblk, OUT_F), lambda i: (i, 0)),
        out_shape=jax.ShapeDtypeStruct((N, OUT_F), f32),
    )(acc20, acc21, v2, dinv, b2, Wo)

    zr = z.reshape(B, T_IN * OUT_F)
    out2d = pl.pallas_call(
        _t3_body,
        grid=(1,),
        in_specs=[
            full2d((B, T_IN * OUT_F)),
            full2d((T_IN * OUT_F, T_OUT * OUT_F)),
            pl.BlockSpec((T_OUT * OUT_F,), lambda i: (0,)),
        ],
        out_specs=full2d((B, T_OUT * OUT_F)),
        out_shape=jax.ShapeDtypeStruct((B, T_OUT * OUT_F), f32),
    )(zr, Wp, bp)
    return out2d.reshape(B, T_OUT, OUT_F)
